# baseline (speedup 1.0000x reference)
import jax
import jax.numpy as jnp
from jax import lax
from jax.experimental import pallas as pl
from jax.experimental.pallas import tpu as pltpu
from jax.experimental.pallas import tpu_sc as plsc

NSEG = 2
NPT = 4096
NPOINT = 2048
NSAMPLE = 32
R2 = 0.16
ROWS, COLS = 16, 512
NTILE = 16
CPT = NPOINT // NTILE
NCH = 64
NROW = NSEG * NPOINT * NSAMPLE
RCHUNK = 4096
NCHUNK = (NPOINT * NSAMPLE) // RCHUNK
GCHUNK = 128


def _fps_body(x_ref, y_ref, z_ref, sm_ref, out_ref, dist_ref):
    rows = lax.broadcasted_iota(jnp.int32, (8, COLS), 0)
    cols = lax.broadcasted_iota(jnp.int32, (8, COLS), 1)
    fi8 = rows * COLS + cols
    big = jnp.full((8, COLS), NPT, jnp.int32)
    x0 = x_ref[:8]
    x1 = x_ref[8:]
    y0 = y_ref[:8]
    y1 = y_ref[8:]
    z0 = z_ref[:8]
    z1 = z_ref[8:]
    dist_ref[...] = jnp.full((ROWS, COLS), 1e10, jnp.float32)

    def step(i, carry):
        f0, f1 = carry
        out_ref[0, i] = f0
        out_ref[1, i] = f1
        cx0 = sm_ref[0, f0]
        cy0 = sm_ref[1, f0]
        cz0 = sm_ref[2, f0]
        cx1 = sm_ref[3, f1]
        cy1 = sm_ref[4, f1]
        cz1 = sm_ref[5, f1]
        dx0 = x0 - cx0
        dy0 = y0 - cy0
        dz0 = z0 - cz0
        d0 = (dx0 * dx0 + dy0 * dy0) + dz0 * dz0
        dx1 = x1 - cx1
        dy1 = y1 - cy1
        dz1 = z1 - cz1
        d1 = (dx1 * dx1 + dy1 * dy1) + dz1 * dz1
        nd0 = jnp.minimum(dist_ref[:8], d0)
        nd1 = jnp.minimum(dist_ref[8:], d1)
        dist_ref[:8] = nd0
        dist_ref[8:] = nd1
        m0 = jnp.max(nd0)
        m1 = jnp.max(nd1)
        i0 = jnp.min(jnp.where(nd0 == m0, fi8, big))
        i1 = jnp.min(jnp.where(nd1 == m1, fi8, big))
        return (i0, i1)

    lax.fori_loop(0, NPOINT, step, (jnp.int32(0), jnp.int32(0)))


def _fps(xyz):
    xs = xyz[0, 0].reshape(ROWS, COLS)
    ys = xyz[0, 1].reshape(ROWS, COLS)
    zs = xyz[0, 2].reshape(ROWS, COLS)
    sm = xyz[0].reshape(3, NSEG, NPT).transpose(1, 0, 2).reshape(6, NPT)
    return pl.pallas_call(
        _fps_body,
        in_specs=[pl.BlockSpec((ROWS, COLS), lambda: (0, 0))
                  for _ in range(3)] + [
                  pl.BlockSpec(memory_space=pltpu.SMEM)],
        out_specs=pl.BlockSpec(memory_space=pltpu.SMEM),
        out_shape=jax.ShapeDtypeStruct((NSEG, NPOINT), jnp.int32),
        scratch_shapes=[pltpu.VMEM((ROWS, COLS), jnp.float32)],
    )(xs, ys, zs, sm)


def _bf16r(v):
    n = plsc.bitcast(v, jnp.uint32)
    r = (n + jnp.uint32(0x7FFF) + ((n >> jnp.uint32(16)) & jnp.uint32(1)))
    return plsc.bitcast(r & jnp.uint32(0xFFFF0000), jnp.float32)


def _sc_body(xyz_hbm, fps_hbm, g16_hbm, table_hbm,
             gath_hbm, lc_hbm,
             xv, yv, zv, xb, yb, zb, pn, fidx, crows, gidx, obuf, rows_v, sem):
    seg = lax.axis_index("c")
    tile = lax.axis_index("s")
    xyzb = seg * (3 * NPT)
    pltpu.sync_copy(xyz_hbm.at[pl.ds(xyzb, NPT)], xv)
    pltpu.sync_copy(xyz_hbm.at[pl.ds(xyzb + NPT, NPT)], yv)
    pltpu.sync_copy(xyz_hbm.at[pl.ds(xyzb + 2 * NPT, NPT)], zv)

    def pbody(i, _):
        s = pl.ds(i * 16, 16)
        px = xv[s]
        py = yv[s]
        pz = zv[s]
        pn[s] = (px * px + py * py) + pz * pz
        xb[s] = _bf16r(px)
        yb[s] = _bf16r(py)
        zb[s] = _bf16r(pz)
        return 0
    lax.fori_loop(0, NPT // 16, pbody, 0)

    lane = lax.iota(jnp.int32, 16)
    segbase = seg * NPT
    pltpu.sync_copy(fps_hbm.at[pl.ds(seg * NPOINT + tile * CPT, CPT)], fidx)

    def fbody(k, _):
        s = pl.ds(k * 16, 16)
        fidx[s] = fidx[s] + segbase
        return 0
    lax.fori_loop(0, CPT // 16, fbody, 0)
    pltpu.async_copy(g16_hbm.at[fidx], crows, sem).wait()
    pltpu.sync_copy(crows, lc_hbm.at[pl.ds(seg * NPOINT + tile * CPT, CPT)])

    def cgroup(k, _):
        for j in range(16):
            crow = crows[k * 16 + j]
            cx = _bf16r(jnp.full((16,), crow[0]))
            cy = _bf16r(jnp.full((16,), crow[1]))
            cz = _bf16r(jnp.full((16,), crow[2]))
            csn = jnp.full((16,), (crow[0] * crow[0] + crow[1] * crow[1])
                           + crow[2] * crow[2])

            def chunk(c, cntv):
                sl = pl.ds(c * 16, 16)
                px = xb[sl]
                py = yb[sl]
                pz = zb[sl]
                pp = pn[sl]
                dot = (cx * px + cy * py) + cz * pz
                d = (csn + pp) - 2.0 * dot
                m = d <= R2
                mi = m.astype(jnp.int32)
                incl = plsc.cumsum(mi)
                pos = (cntv + incl) - mi
                keep = m & (pos < NSAMPLE)
                pidx = lane + c * 16
                plsc.store_scatter(obuf, [pos], pidx, mask=keep)
                return cntv + plsc.all_reduce_population_count(m)

            def body(c2, cntv):
                return chunk(2 * c2 + 1, chunk(2 * c2, cntv))

            cntv = lax.fori_loop(0, NPT // 32, body,
                                 jnp.zeros((16,), jnp.int32))
            cnt = cntv[0]
            nz = jnp.minimum(cnt, 1)
            first = nz * obuf[pl.ds(0, 16)][0] + (1 - nz) * (NPT - 1)
            sbase = (k * 16 + j) * NSAMPLE
            for kk in range(NSAMPLE // 16):
                cur = obuf[pl.ds(kk * 16, 16)]
                sl_ids = lane + kk * 16
                val = jnp.where(sl_ids < cnt, cur, jnp.full((16,), first))
                gidx[pl.ds(sbase + kk * 16, 16)] = val + segbase
        return 0
    lax.fori_loop(0, CPT // 16, cgroup, 0)

    out_base = seg * (NPOINT * NSAMPLE) + tile * (CPT * NSAMPLE)

    def hbody(c, _):
        idx_sl = gidx.at[pl.ds(c * GCHUNK, GCHUNK)]
        pltpu.async_copy(table_hbm.at[idx_sl], rows_v, sem).wait()
        pltpu.sync_copy(rows_v, gath_hbm.at[pl.ds(out_base + c * GCHUNK, GCHUNK)])
        return 0
    lax.fori_loop(0, (CPT * NSAMPLE) // GCHUNK, hbody, 0)


def _sc_stage(xyz2, fps, g16, table):
    mesh = plsc.VectorSubcoreMesh(core_axis_name="c", subcore_axis_name="s",
                                  num_cores=2, num_subcores=16)
    f = pl.kernel(
        _sc_body,
        out_type=(
            jax.ShapeDtypeStruct((NROW, NCH), jnp.float32),
            jax.ShapeDtypeStruct((NSEG * NPOINT, 16), jnp.float32),
        ),
        mesh=mesh,
        compiler_params=pltpu.CompilerParams(use_tc_tiling_on_sc=False,
                                             needs_layout_passes=False),
        scratch_types=[
            pltpu.VMEM((NPT,), jnp.float32),
            pltpu.VMEM((NPT,), jnp.float32),
            pltpu.VMEM((NPT,), jnp.float32),
            pltpu.VMEM((NPT,), jnp.float32),
            pltpu.VMEM((NPT,), jnp.float32),
            pltpu.VMEM((NPT,), jnp.float32),
            pltpu.VMEM((NPT,), jnp.float32),
            pltpu.VMEM((CPT,), jnp.int32),
            pltpu.VMEM((CPT, 16), jnp.float32),
            pltpu.VMEM((CPT * NSAMPLE,), jnp.int32),
            pltpu.VMEM((NSAMPLE,), jnp.int32),
            pltpu.VMEM((GCHUNK, NCH), jnp.float32),
            pltpu.SemaphoreType.DMA,
        ],
    )
    return f(xyz2, fps, g16, table)


def _l0_body(g_ref, nx_ref, w_ref, b_ref, y_ref, s_ref, q_ref):
    x = g_ref[...] - nx_ref[...]
    y = lax.dot_general(x, w_ref[0], (((1,), (1,)), ((), ())),
                        preferred_element_type=jnp.float32) + b_ref[0]
    y_ref[...] = y

    @pl.when(pl.program_id(1) == 0)
    def _():
        s_ref[...] = jnp.zeros_like(s_ref)
        q_ref[...] = jnp.zeros_like(q_ref)
    s_ref[...] += jnp.sum(y, axis=0).reshape(1, 1, -1)
    q_ref[...] += jnp.sum(y * y, axis=0).reshape(1, 1, -1)


def _bn_relu(y, s_ref, q_ref, g_ref, be_ref):
    n = jnp.float32(NPOINT * NSAMPLE)
    mu = s_ref[0, 0] / n
    var = q_ref[0, 0] / n - mu * mu
    rstd = lax.rsqrt(var + 1e-5)
    scale = g_ref[0, 0] * rstd
    shift = be_ref[0, 0] - mu * scale
    return jnp.maximum(y * scale[None, :] + shift[None, :], 0.0)


def _lk_body(y_ref, s_ref, q_ref, g_ref, be_ref, w_ref, b_ref,
             o_ref, s2_ref, q2_ref):
    x = _bn_relu(y_ref[...], s_ref, q_ref, g_ref, be_ref)
    y = lax.dot_general(x, w_ref[0], (((1,), (1,)), ((), ())),
                        preferred_element_type=jnp.float32) + b_ref[0]
    o_ref[...] = y

    @pl.when(pl.program_id(1) == 0)
    def _():
        s2_ref[...] = jnp.zeros_like(s2_ref)
        q2_ref[...] = jnp.zeros_like(q2_ref)
    s2_ref[...] += jnp.sum(y, axis=0).reshape(1, 1, -1)
    q2_ref[...] += jnp.sum(y * y, axis=0).reshape(1, 1, -1)


def _fin_body(y_ref, s_ref, q_ref, g_ref, be_ref, o_ref):
    x = _bn_relu(y_ref[...], s_ref, q_ref, g_ref, be_ref)
    x = x.reshape(RCHUNK // NSAMPLE, NSAMPLE, x.shape[-1])
    o_ref[...] = jnp.max(x, axis=1)[None]


def _row_spec(c):
    return pl.BlockSpec((RCHUNK, c), lambda s, i: (s * NCHUNK + i, 0))


def _seg_spec(shape):
    return pl.BlockSpec((1,) + shape[1:], lambda s, i: (s,) + (0,) * (len(shape) - 1))


def _stat_spec(c):
    return pl.BlockSpec((1, 1, c), lambda s, i: (s, 0, 0))


def _mlp(gath, nx, lxyz, w, b, g, be):
    grid = (NSEG, NCHUNK)
    y0, s0, q0 = pl.pallas_call(
        _l0_body,
        grid=grid,
        in_specs=[_row_spec(NCH), _row_spec(NCH),
                  _seg_spec(w[0].shape), _seg_spec(b[0].shape)],
        out_specs=[_row_spec(64), _stat_spec(64), _stat_spec(64)],
        out_shape=[jax.ShapeDtypeStruct((NROW, 64), jnp.float32),
                   jax.ShapeDtypeStruct((NSEG, 1, 64), jnp.float32),
                   jax.ShapeDtypeStruct((NSEG, 1, 64), jnp.float32)],
    )(gath, nx, w[0], b[0])

    def layer(yk, sk, qk, l, co):
        return pl.pallas_call(
            _lk_body,
            grid=grid,
            in_specs=[_row_spec(yk.shape[-1]),
                      _stat_spec(yk.shape[-1]), _stat_spec(yk.shape[-1]),
                      _seg_spec(g[l - 1].shape), _seg_spec(be[l - 1].shape),
                      _seg_spec(w[l].shape), _seg_spec(b[l].shape)],
            out_specs=[_row_spec(co), _stat_spec(co), _stat_spec(co)],
            out_shape=[jax.ShapeDtypeStruct((NROW, co), jnp.float32),
                       jax.ShapeDtypeStruct((NSEG, 1, co), jnp.float32),
                       jax.ShapeDtypeStruct((NSEG, 1, co), jnp.float32)],
        )(yk, sk, qk, g[l - 1], be[l - 1], w[l], b[l])

    y1, s1, q1 = layer(y0, s0, q0, 1, 64)
    y2, s2, q2 = layer(y1, s1, q1, 2, 128)

    out = pl.pallas_call(
        _fin_body,
        grid=grid,
        in_specs=[_row_spec(128), _stat_spec(128), _stat_spec(128),
                  _seg_spec(g[2].shape), _seg_spec(be[2].shape)],
        out_specs=pl.BlockSpec((1, RCHUNK // NSAMPLE, 128),
                               lambda s, i: (s, i, 0)),
        out_shape=jax.ShapeDtypeStruct((NSEG, NPOINT, 128), jnp.float32),
    )(y2, s2, q2, g[2], be[2])
    return out


def kernel(xy, xyz, points, point_split, conv_w0, conv_b0, conv_w1, conv_b1,
           conv_w2, conv_b2, bn_g0, bn_b0, bn_g1, bn_b1, bn_g2, bn_b2):
    xy = xy + point_split[0].astype(xy.dtype)
    fps = _fps(xyz)

    xyz2 = xyz[0].reshape(3, NSEG, NPT).transpose(1, 0, 2)
    table = jnp.concatenate([xyz[0].T, points[0].T], axis=1)
    g16 = jnp.concatenate([xyz[0].T, xy[0].T,
                           jnp.zeros((NSEG * NPT, 11), jnp.float32)], axis=1)
    gath, lc = _sc_stage(xyz2.reshape(-1), fps.reshape(-1), g16, table)
    lc = lc.reshape(NSEG, NPOINT, 16)
    lxyz = lc[:, :, 0:3].transpose(0, 2, 1)
    lxy = lc[:, :, 3:5].transpose(0, 2, 1)

    nx3 = jnp.broadcast_to(lxyz.transpose(0, 2, 1)[:, :, None, :],
                           (NSEG, NPOINT, NSAMPLE, 3)).reshape(NROW, 3)
    nx = jnp.concatenate(
        [nx3, jnp.zeros((NROW, NCH - 3), jnp.float32)], axis=1)

    w = (conv_w0, conv_w1, conv_w2)
    b = tuple(x.reshape(NSEG, 1, -1) for x in (conv_b0, conv_b1, conv_b2))
    g = tuple(x.reshape(NSEG, 1, -1) for x in (bn_g0, bn_g1, bn_g2))
    be = tuple(x.reshape(NSEG, 1, -1) for x in (bn_b0, bn_b1, bn_b2))
    out = _mlp(gath, nx, lxyz, w, b, g, be)

    new_xy = jnp.concatenate([lxy[0], lxy[1]], axis=-1)[None]
    new_xyz = jnp.concatenate([lxyz[0], lxyz[1]], axis=-1)[None]
    new_pts = jnp.concatenate([out[0].T, out[1].T], axis=-1)[None]
    split = jnp.array([0, NPOINT, 2 * NPOINT], dtype=jnp.int32)
    return new_xy, new_xyz, new_pts, split


# speedup vs baseline: 1.0002x; 1.0002x over previous
import jax
import jax.numpy as jnp
from jax import lax
from jax.experimental import pallas as pl
from jax.experimental.pallas import tpu as pltpu
from jax.experimental.pallas import tpu_sc as plsc

NSEG = 2
NPT = 4096
NPOINT = 2048
NSAMPLE = 32
R2 = 0.16
ROWS, COLS = 16, 512
NTILE = 16
CPT = NPOINT // NTILE
NCH = 64
NROW = NSEG * NPOINT * NSAMPLE
RCHUNK = 4096
NCHUNK = (NPOINT * NSAMPLE) // RCHUNK
GCHUNK = 128


def _fps_body(x_ref, y_ref, z_ref, sm_ref, out_ref):
    rows = lax.broadcasted_iota(jnp.int32, (8, COLS), 0)
    cols = lax.broadcasted_iota(jnp.int32, (8, COLS), 1)
    fi8 = rows * COLS + cols
    big = jnp.full((8, COLS), NPT, jnp.int32)
    x0 = x_ref[:8]
    x1 = x_ref[8:]
    y0 = y_ref[:8]
    y1 = y_ref[8:]
    z0 = z_ref[:8]
    z1 = z_ref[8:]
    inf8 = jnp.full((8, COLS), 1e10, jnp.float32)

    def step(i, carry):
        f0, f1, pd0, pd1 = carry
        out_ref[0, i] = f0
        out_ref[1, i] = f1
        cx0 = sm_ref[0, f0]
        cy0 = sm_ref[1, f0]
        cz0 = sm_ref[2, f0]
        cx1 = sm_ref[3, f1]
        cy1 = sm_ref[4, f1]
        cz1 = sm_ref[5, f1]
        dx0 = x0 - cx0
        dy0 = y0 - cy0
        dz0 = z0 - cz0
        d0 = (dx0 * dx0 + dy0 * dy0) + dz0 * dz0
        dx1 = x1 - cx1
        dy1 = y1 - cy1
        dz1 = z1 - cz1
        d1 = (dx1 * dx1 + dy1 * dy1) + dz1 * dz1
        nd0 = jnp.minimum(pd0, d0)
        nd1 = jnp.minimum(pd1, d1)
        m0 = jnp.max(nd0)
        m1 = jnp.max(nd1)
        i0 = jnp.min(jnp.where(nd0 == m0, fi8, big))
        i1 = jnp.min(jnp.where(nd1 == m1, fi8, big))
        return (i0, i1, nd0, nd1)

    lax.fori_loop(0, NPOINT, step,
                  (jnp.int32(0), jnp.int32(0), inf8, inf8))


def _fps(xyz):
    xs = xyz[0, 0].reshape(ROWS, COLS)
    ys = xyz[0, 1].reshape(ROWS, COLS)
    zs = xyz[0, 2].reshape(ROWS, COLS)
    sm = xyz[0].reshape(3, NSEG, NPT).transpose(1, 0, 2).reshape(6, NPT)
    return pl.pallas_call(
        _fps_body,
        in_specs=[pl.BlockSpec((ROWS, COLS), lambda: (0, 0))
                  for _ in range(3)] + [
                  pl.BlockSpec(memory_space=pltpu.SMEM)],
        out_specs=pl.BlockSpec(memory_space=pltpu.SMEM),
        out_shape=jax.ShapeDtypeStruct((NSEG, NPOINT), jnp.int32),
    )(xs, ys, zs, sm)


def _bf16r(v):
    n = plsc.bitcast(v, jnp.uint32)
    r = (n + jnp.uint32(0x7FFF) + ((n >> jnp.uint32(16)) & jnp.uint32(1)))
    return plsc.bitcast(r & jnp.uint32(0xFFFF0000), jnp.float32)


def _sc_body(xyz_hbm, fps_hbm, g16_hbm, table_hbm,
             gath_hbm, lc_hbm,
             xv, yv, zv, xb, yb, zb, pn, fidx, crows, gidx, obuf, rows_v, sem):
    seg = lax.axis_index("c")
    tile = lax.axis_index("s")
    xyzb = seg * (3 * NPT)
    pltpu.sync_copy(xyz_hbm.at[pl.ds(xyzb, NPT)], xv)
    pltpu.sync_copy(xyz_hbm.at[pl.ds(xyzb + NPT, NPT)], yv)
    pltpu.sync_copy(xyz_hbm.at[pl.ds(xyzb + 2 * NPT, NPT)], zv)

    def pbody(i, _):
        s = pl.ds(i * 16, 16)
        px = xv[s]
        py = yv[s]
        pz = zv[s]
        pn[s] = (px * px + py * py) + pz * pz
        xb[s] = _bf16r(px)
        yb[s] = _bf16r(py)
        zb[s] = _bf16r(pz)
        return 0
    lax.fori_loop(0, NPT // 16, pbody, 0)

    lane = lax.iota(jnp.int32, 16)
    segbase = seg * NPT
    pltpu.sync_copy(fps_hbm.at[pl.ds(seg * NPOINT + tile * CPT, CPT)], fidx)

    def fbody(k, _):
        s = pl.ds(k * 16, 16)
        fidx[s] = fidx[s] + segbase
        return 0
    lax.fori_loop(0, CPT // 16, fbody, 0)
    pltpu.async_copy(g16_hbm.at[fidx], crows, sem).wait()
    pltpu.sync_copy(crows, lc_hbm.at[pl.ds(seg * NPOINT + tile * CPT, CPT)])

    def cgroup(k, _):
        for j in range(16):
            crow = crows[k * 16 + j]
            cx = _bf16r(jnp.full((16,), crow[0]))
            cy = _bf16r(jnp.full((16,), crow[1]))
            cz = _bf16r(jnp.full((16,), crow[2]))
            csn = jnp.full((16,), (crow[0] * crow[0] + crow[1] * crow[1])
                           + crow[2] * crow[2])

            def chunk(c, cntv):
                sl = pl.ds(c * 16, 16)
                px = xb[sl]
                py = yb[sl]
                pz = zb[sl]
                pp = pn[sl]
                dot = (cx * px + cy * py) + cz * pz
                d = (csn + pp) - 2.0 * dot
                m = d <= R2
                mi = m.astype(jnp.int32)
                incl = plsc.cumsum(mi)
                pos = (cntv + incl) - mi
                keep = m & (pos < NSAMPLE)
                pidx = lane + c * 16
                plsc.store_scatter(obuf, [pos], pidx, mask=keep)
                return cntv + plsc.all_reduce_population_count(m)

            def body(c2, cntv):
                return chunk(2 * c2 + 1, chunk(2 * c2, cntv))

            cntv = lax.fori_loop(0, NPT // 32, body,
                                 jnp.zeros((16,), jnp.int32))
            cnt = cntv[0]
            nz = jnp.minimum(cnt, 1)
            first = nz * obuf[pl.ds(0, 16)][0] + (1 - nz) * (NPT - 1)
            sbase = (k * 16 + j) * NSAMPLE
            for kk in range(NSAMPLE // 16):
                cur = obuf[pl.ds(kk * 16, 16)]
                sl_ids = lane + kk * 16
                val = jnp.where(sl_ids < cnt, cur, jnp.full((16,), first))
                gidx[pl.ds(sbase + kk * 16, 16)] = val + segbase
        return 0
    lax.fori_loop(0, CPT // 16, cgroup, 0)

    out_base = seg * (NPOINT * NSAMPLE) + tile * (CPT * NSAMPLE)

    def hbody(c, _):
        idx_sl = gidx.at[pl.ds(c * GCHUNK, GCHUNK)]
        pltpu.async_copy(table_hbm.at[idx_sl], rows_v, sem).wait()
        pltpu.sync_copy(rows_v, gath_hbm.at[pl.ds(out_base + c * GCHUNK, GCHUNK)])
        return 0
    lax.fori_loop(0, (CPT * NSAMPLE) // GCHUNK, hbody, 0)


def _sc_stage(xyz2, fps, g16, table):
    mesh = plsc.VectorSubcoreMesh(core_axis_name="c", subcore_axis_name="s",
                                  num_cores=2, num_subcores=16)
    f = pl.kernel(
        _sc_body,
        out_type=(
            jax.ShapeDtypeStruct((NROW, NCH), jnp.float32),
            jax.ShapeDtypeStruct((NSEG * NPOINT, 16), jnp.float32),
        ),
        mesh=mesh,
        compiler_params=pltpu.CompilerParams(use_tc_tiling_on_sc=False,
                                             needs_layout_passes=False),
        scratch_types=[
            pltpu.VMEM((NPT,), jnp.float32),
            pltpu.VMEM((NPT,), jnp.float32),
            pltpu.VMEM((NPT,), jnp.float32),
            pltpu.VMEM((NPT,), jnp.float32),
            pltpu.VMEM((NPT,), jnp.float32),
            pltpu.VMEM((NPT,), jnp.float32),
            pltpu.VMEM((NPT,), jnp.float32),
            pltpu.VMEM((CPT,), jnp.int32),
            pltpu.VMEM((CPT, 16), jnp.float32),
            pltpu.VMEM((CPT * NSAMPLE,), jnp.int32),
            pltpu.VMEM((NSAMPLE,), jnp.int32),
            pltpu.VMEM((GCHUNK, NCH), jnp.float32),
            pltpu.SemaphoreType.DMA,
        ],
    )
    return f(xyz2, fps, g16, table)


def _l0_body(g_ref, nx_ref, w_ref, b_ref, y_ref, s_ref, q_ref):
    x = g_ref[...] - nx_ref[...]
    y = lax.dot_general(x, w_ref[0], (((1,), (1,)), ((), ())),
                        preferred_element_type=jnp.float32) + b_ref[0]
    y_ref[...] = y

    @pl.when(pl.program_id(1) == 0)
    def _():
        s_ref[...] = jnp.zeros_like(s_ref)
        q_ref[...] = jnp.zeros_like(q_ref)
    s_ref[...] += jnp.sum(y, axis=0).reshape(1, 1, -1)
    q_ref[...] += jnp.sum(y * y, axis=0).reshape(1, 1, -1)


def _bn_relu(y, s_ref, q_ref, g_ref, be_ref):
    n = jnp.float32(NPOINT * NSAMPLE)
    mu = s_ref[0, 0] / n
    var = q_ref[0, 0] / n - mu * mu
    rstd = lax.rsqrt(var + 1e-5)
    scale = g_ref[0, 0] * rstd
    shift = be_ref[0, 0] - mu * scale
    return jnp.maximum(y * scale[None, :] + shift[None, :], 0.0)


def _lk_body(y_ref, s_ref, q_ref, g_ref, be_ref, w_ref, b_ref,
             o_ref, s2_ref, q2_ref):
    x = _bn_relu(y_ref[...], s_ref, q_ref, g_ref, be_ref)
    y = lax.dot_general(x, w_ref[0], (((1,), (1,)), ((), ())),
                        preferred_element_type=jnp.float32) + b_ref[0]
    o_ref[...] = y

    @pl.when(pl.program_id(1) == 0)
    def _():
        s2_ref[...] = jnp.zeros_like(s2_ref)
        q2_ref[...] = jnp.zeros_like(q2_ref)
    s2_ref[...] += jnp.sum(y, axis=0).reshape(1, 1, -1)
    q2_ref[...] += jnp.sum(y * y, axis=0).reshape(1, 1, -1)


def _fin_body(y_ref, s_ref, q_ref, g_ref, be_ref, o_ref):
    x = _bn_relu(y_ref[...], s_ref, q_ref, g_ref, be_ref)
    x = x.reshape(RCHUNK // NSAMPLE, NSAMPLE, x.shape[-1])
    o_ref[...] = jnp.max(x, axis=1)[None]


def _row_spec(c):
    return pl.BlockSpec((RCHUNK, c), lambda s, i: (s * NCHUNK + i, 0))


def _seg_spec(shape):
    return pl.BlockSpec((1,) + shape[1:], lambda s, i: (s,) + (0,) * (len(shape) - 1))


def _stat_spec(c):
    return pl.BlockSpec((1, 1, c), lambda s, i: (s, 0, 0))


def _mlp(gath, nx, lxyz, w, b, g, be):
    grid = (NSEG, NCHUNK)
    y0, s0, q0 = pl.pallas_call(
        _l0_body,
        grid=grid,
        in_specs=[_row_spec(NCH), _row_spec(NCH),
                  _seg_spec(w[0].shape), _seg_spec(b[0].shape)],
        out_specs=[_row_spec(64), _stat_spec(64), _stat_spec(64)],
        out_shape=[jax.ShapeDtypeStruct((NROW, 64), jnp.float32),
                   jax.ShapeDtypeStruct((NSEG, 1, 64), jnp.float32),
                   jax.ShapeDtypeStruct((NSEG, 1, 64), jnp.float32)],
    )(gath, nx, w[0], b[0])

    def layer(yk, sk, qk, l, co):
        return pl.pallas_call(
            _lk_body,
            grid=grid,
            in_specs=[_row_spec(yk.shape[-1]),
                      _stat_spec(yk.shape[-1]), _stat_spec(yk.shape[-1]),
                      _seg_spec(g[l - 1].shape), _seg_spec(be[l - 1].shape),
                      _seg_spec(w[l].shape), _seg_spec(b[l].shape)],
            out_specs=[_row_spec(co), _stat_spec(co), _stat_spec(co)],
            out_shape=[jax.ShapeDtypeStruct((NROW, co), jnp.float32),
                       jax.ShapeDtypeStruct((NSEG, 1, co), jnp.float32),
                       jax.ShapeDtypeStruct((NSEG, 1, co), jnp.float32)],
        )(yk, sk, qk, g[l - 1], be[l - 1], w[l], b[l])

    y1, s1, q1 = layer(y0, s0, q0, 1, 64)
    y2, s2, q2 = layer(y1, s1, q1, 2, 128)

    out = pl.pallas_call(
        _fin_body,
        grid=grid,
        in_specs=[_row_spec(128), _stat_spec(128), _stat_spec(128),
                  _seg_spec(g[2].shape), _seg_spec(be[2].shape)],
        out_specs=pl.BlockSpec((1, RCHUNK // NSAMPLE, 128),
                               lambda s, i: (s, i, 0)),
        out_shape=jax.ShapeDtypeStruct((NSEG, NPOINT, 128), jnp.float32),
    )(y2, s2, q2, g[2], be[2])
    return out


def kernel(xy, xyz, points, point_split, conv_w0, conv_b0, conv_w1, conv_b1,
           conv_w2, conv_b2, bn_g0, bn_b0, bn_g1, bn_b1, bn_g2, bn_b2):
    xy = xy + point_split[0].astype(xy.dtype)
    fps = _fps(xyz)

    xyz2 = xyz[0].reshape(3, NSEG, NPT).transpose(1, 0, 2)
    table = jnp.concatenate([xyz[0].T, points[0].T], axis=1)
    g16 = jnp.concatenate([xyz[0].T, xy[0].T,
                           jnp.zeros((NSEG * NPT, 11), jnp.float32)], axis=1)
    gath, lc = _sc_stage(xyz2.reshape(-1), fps.reshape(-1), g16, table)
    lc = lc.reshape(NSEG, NPOINT, 16)
    lxyz = lc[:, :, 0:3].transpose(0, 2, 1)
    lxy = lc[:, :, 3:5].transpose(0, 2, 1)

    nx3 = jnp.broadcast_to(lxyz.transpose(0, 2, 1)[:, :, None, :],
                           (NSEG, NPOINT, NSAMPLE, 3)).reshape(NROW, 3)
    nx = jnp.concatenate(
        [nx3, jnp.zeros((NROW, NCH - 3), jnp.float32)], axis=1)

    w = (conv_w0, conv_w1, conv_w2)
    b = tuple(x.reshape(NSEG, 1, -1) for x in (conv_b0, conv_b1, conv_b2))
    g = tuple(x.reshape(NSEG, 1, -1) for x in (bn_g0, bn_g1, bn_g2))
    be = tuple(x.reshape(NSEG, 1, -1) for x in (bn_b0, bn_b1, bn_b2))
    out = _mlp(gath, nx, lxyz, w, b, g, be)

    new_xy = jnp.concatenate([lxy[0], lxy[1]], axis=-1)[None]
    new_xyz = jnp.concatenate([lxyz[0], lxyz[1]], axis=-1)[None]
    new_pts = jnp.concatenate([out[0].T, out[1].T], axis=-1)[None]
    split = jnp.array([0, NPOINT, 2 * NPOINT], dtype=jnp.int32)
    return new_xy, new_xyz, new_pts, split


# speedup vs baseline: 1.0168x; 1.0166x over previous
import jax
import jax.numpy as jnp
from jax import lax
from jax.experimental import pallas as pl
from jax.experimental.pallas import tpu as pltpu
from jax.experimental.pallas import tpu_sc as plsc

NSEG = 2
NPT = 4096
NPOINT = 2048
NSAMPLE = 32
R2 = 0.16
ROWS, COLS = 16, 512
NTILE = 16
CPT = NPOINT // NTILE
NCH = 64
NROW = NSEG * NPOINT * NSAMPLE
RCHUNK = 4096
NCHUNK = (NPOINT * NSAMPLE) // RCHUNK
GCHUNK = 128


def _fps_body(x_ref, y_ref, z_ref, sm_ref, out_ref):
    rows = lax.broadcasted_iota(jnp.int32, (8, COLS), 0)
    cols = lax.broadcasted_iota(jnp.int32, (8, COLS), 1)
    fi8 = rows * COLS + cols
    big = jnp.full((8, COLS), NPT, jnp.int32)
    x0 = x_ref[:8]
    x1 = x_ref[8:]
    y0 = y_ref[:8]
    y1 = y_ref[8:]
    z0 = z_ref[:8]
    z1 = z_ref[8:]
    inf8 = jnp.full((8, COLS), 1e10, jnp.float32)

    def step(i, carry):
        f0, f1, pd0, pd1 = carry
        out_ref[0, i] = f0
        out_ref[1, i] = f1
        cx0 = sm_ref[0, f0]
        cy0 = sm_ref[1, f0]
        cz0 = sm_ref[2, f0]
        cx1 = sm_ref[3, f1]
        cy1 = sm_ref[4, f1]
        cz1 = sm_ref[5, f1]
        dx0 = x0 - cx0
        dy0 = y0 - cy0
        dz0 = z0 - cz0
        d0 = (dx0 * dx0 + dy0 * dy0) + dz0 * dz0
        dx1 = x1 - cx1
        dy1 = y1 - cy1
        dz1 = z1 - cz1
        d1 = (dx1 * dx1 + dy1 * dy1) + dz1 * dz1
        nd0 = jnp.minimum(pd0, d0)
        nd1 = jnp.minimum(pd1, d1)
        m0 = jnp.max(nd0, axis=(0, 1), keepdims=True)
        m1 = jnp.max(nd1, axis=(0, 1), keepdims=True)
        i0 = jnp.min(jnp.where(nd0 == m0, fi8, big))
        i1 = jnp.min(jnp.where(nd1 == m1, fi8, big))
        return (i0, i1, nd0, nd1)

    lax.fori_loop(0, NPOINT, step,
                  (jnp.int32(0), jnp.int32(0), inf8, inf8))


def _fps(xyz):
    xs = xyz[0, 0].reshape(ROWS, COLS)
    ys = xyz[0, 1].reshape(ROWS, COLS)
    zs = xyz[0, 2].reshape(ROWS, COLS)
    sm = xyz[0].reshape(3, NSEG, NPT).transpose(1, 0, 2).reshape(6, NPT)
    return pl.pallas_call(
        _fps_body,
        in_specs=[pl.BlockSpec((ROWS, COLS), lambda: (0, 0))
                  for _ in range(3)] + [
                  pl.BlockSpec(memory_space=pltpu.SMEM)],
        out_specs=pl.BlockSpec(memory_space=pltpu.SMEM),
        out_shape=jax.ShapeDtypeStruct((NSEG, NPOINT), jnp.int32),
    )(xs, ys, zs, sm)


def _bf16r(v):
    n = plsc.bitcast(v, jnp.uint32)
    r = (n + jnp.uint32(0x7FFF) + ((n >> jnp.uint32(16)) & jnp.uint32(1)))
    return plsc.bitcast(r & jnp.uint32(0xFFFF0000), jnp.float32)


def _sc_body(xyz_hbm, fps_hbm, g16_hbm, table_hbm,
             gath_hbm, lc_hbm,
             xv, yv, zv, xb, yb, zb, pn, fidx, crows, gidx, obuf,
             rows_v, rows_w, sem, sem2):
    seg = lax.axis_index("c")
    tile = lax.axis_index("s")
    xyzb = seg * (3 * NPT)
    pltpu.sync_copy(xyz_hbm.at[pl.ds(xyzb, NPT)], xv)
    pltpu.sync_copy(xyz_hbm.at[pl.ds(xyzb + NPT, NPT)], yv)
    pltpu.sync_copy(xyz_hbm.at[pl.ds(xyzb + 2 * NPT, NPT)], zv)

    def pbody(i, _):
        s = pl.ds(i * 16, 16)
        px = xv[s]
        py = yv[s]
        pz = zv[s]
        pn[s] = (px * px + py * py) + pz * pz
        xb[s] = _bf16r(px)
        yb[s] = _bf16r(py)
        zb[s] = _bf16r(pz)
        return 0
    lax.fori_loop(0, NPT // 16, pbody, 0)

    lane = lax.iota(jnp.int32, 16)
    segbase = seg * NPT
    pltpu.sync_copy(fps_hbm.at[pl.ds(seg * NPOINT + tile * CPT, CPT)], fidx)

    def fbody(k, _):
        s = pl.ds(k * 16, 16)
        fidx[s] = fidx[s] + segbase
        return 0
    lax.fori_loop(0, CPT // 16, fbody, 0)
    pltpu.async_copy(g16_hbm.at[fidx], crows, sem).wait()
    pltpu.sync_copy(crows, lc_hbm.at[pl.ds(seg * NPOINT + tile * CPT, CPT)])

    def cgroup(k, _):
        for j in range(16):
            crow = crows[k * 16 + j]
            cx = _bf16r(jnp.full((16,), crow[0]))
            cy = _bf16r(jnp.full((16,), crow[1]))
            cz = _bf16r(jnp.full((16,), crow[2]))
            csn = jnp.full((16,), (crow[0] * crow[0] + crow[1] * crow[1])
                           + crow[2] * crow[2])

            def chunk(c, cntv):
                sl = pl.ds(c * 16, 16)
                px = xb[sl]
                py = yb[sl]
                pz = zb[sl]
                pp = pn[sl]
                dot = (cx * px + cy * py) + cz * pz
                d = (csn + pp) - 2.0 * dot
                m = d <= R2
                mi = m.astype(jnp.int32)
                incl = plsc.cumsum(mi)
                pos = (cntv + incl) - mi
                keep = m & (pos < NSAMPLE)
                pidx = lane + c * 16
                plsc.store_scatter(obuf, [pos], pidx, mask=keep)
                return cntv + plsc.all_reduce_population_count(m)

            def body(c2, cntv):
                return chunk(2 * c2 + 1, chunk(2 * c2, cntv))

            cntv = lax.fori_loop(0, NPT // 32, body,
                                 jnp.zeros((16,), jnp.int32))
            cnt = cntv[0]
            nz = jnp.minimum(cnt, 1)
            first = nz * obuf[pl.ds(0, 16)][0] + (1 - nz) * (NPT - 1)
            sbase = (k * 16 + j) * NSAMPLE
            for kk in range(NSAMPLE // 16):
                cur = obuf[pl.ds(kk * 16, 16)]
                sl_ids = lane + kk * 16
                val = jnp.where(sl_ids < cnt, cur, jnp.full((16,), first))
                gidx[pl.ds(sbase + kk * 16, 16)] = val + segbase
        return 0
    lax.fori_loop(0, CPT // 16, cgroup, 0)

    out_base = seg * (NPOINT * NSAMPLE) + tile * (CPT * NSAMPLE)
    nchunk = (CPT * NSAMPLE) // GCHUNK
    bufs = (rows_v, rows_w)
    sems = (sem, sem2)

    def _g(c):
        return pltpu.async_copy(
            table_hbm.at[gidx.at[pl.ds(c * GCHUNK, GCHUNK)]],
            bufs[c % 2], sems[c % 2])

    d = _g(0)
    for c in range(nchunk):
        dn = _g(c + 1) if c + 1 < nchunk else None
        d.wait()
        pltpu.sync_copy(bufs[c % 2],
                        gath_hbm.at[pl.ds(out_base + c * GCHUNK, GCHUNK)])
        d = dn


def _sc_stage(xyz2, fps, g16, table):
    mesh = plsc.VectorSubcoreMesh(core_axis_name="c", subcore_axis_name="s",
                                  num_cores=2, num_subcores=16)
    f = pl.kernel(
        _sc_body,
        out_type=(
            jax.ShapeDtypeStruct((NROW, NCH), jnp.float32),
            jax.ShapeDtypeStruct((NSEG * NPOINT, 16), jnp.float32),
        ),
        mesh=mesh,
        compiler_params=pltpu.CompilerParams(use_tc_tiling_on_sc=False,
                                             needs_layout_passes=False),
        scratch_types=[
            pltpu.VMEM((NPT,), jnp.float32),
            pltpu.VMEM((NPT,), jnp.float32),
            pltpu.VMEM((NPT,), jnp.float32),
            pltpu.VMEM((NPT,), jnp.float32),
            pltpu.VMEM((NPT,), jnp.float32),
            pltpu.VMEM((NPT,), jnp.float32),
            pltpu.VMEM((NPT,), jnp.float32),
            pltpu.VMEM((CPT,), jnp.int32),
            pltpu.VMEM((CPT, 16), jnp.float32),
            pltpu.VMEM((CPT * NSAMPLE,), jnp.int32),
            pltpu.VMEM((NSAMPLE,), jnp.int32),
            pltpu.VMEM((GCHUNK, NCH), jnp.float32),
            pltpu.VMEM((GCHUNK, NCH), jnp.float32),
            pltpu.SemaphoreType.DMA,
            pltpu.SemaphoreType.DMA,
        ],
    )
    return f(xyz2, fps, g16, table)


def _l0_body(g_ref, nx_ref, w_ref, b_ref, y_ref, s_ref, q_ref):
    x = g_ref[...] - nx_ref[...]
    y = lax.dot_general(x, w_ref[0], (((1,), (1,)), ((), ())),
                        preferred_element_type=jnp.float32) + b_ref[0]
    y_ref[...] = y

    @pl.when(pl.program_id(1) == 0)
    def _():
        s_ref[...] = jnp.zeros_like(s_ref)
        q_ref[...] = jnp.zeros_like(q_ref)
    s_ref[...] += jnp.sum(y, axis=0).reshape(1, 1, -1)
    q_ref[...] += jnp.sum(y * y, axis=0).reshape(1, 1, -1)


def _bn_relu(y, s_ref, q_ref, g_ref, be_ref):
    n = jnp.float32(NPOINT * NSAMPLE)
    mu = s_ref[0, 0] / n
    var = q_ref[0, 0] / n - mu * mu
    rstd = lax.rsqrt(var + 1e-5)
    scale = g_ref[0, 0] * rstd
    shift = be_ref[0, 0] - mu * scale
    return jnp.maximum(y * scale[None, :] + shift[None, :], 0.0)


def _lk_body(y_ref, s_ref, q_ref, g_ref, be_ref, w_ref, b_ref,
             o_ref, s2_ref, q2_ref):
    x = _bn_relu(y_ref[...], s_ref, q_ref, g_ref, be_ref)
    y = lax.dot_general(x, w_ref[0], (((1,), (1,)), ((), ())),
                        preferred_element_type=jnp.float32) + b_ref[0]
    o_ref[...] = y

    @pl.when(pl.program_id(1) == 0)
    def _():
        s2_ref[...] = jnp.zeros_like(s2_ref)
        q2_ref[...] = jnp.zeros_like(q2_ref)
    s2_ref[...] += jnp.sum(y, axis=0).reshape(1, 1, -1)
    q2_ref[...] += jnp.sum(y * y, axis=0).reshape(1, 1, -1)


def _fin_body(y_ref, s_ref, q_ref, g_ref, be_ref, o_ref):
    x = _bn_relu(y_ref[...], s_ref, q_ref, g_ref, be_ref)
    x = x.reshape(RCHUNK // NSAMPLE, NSAMPLE, x.shape[-1])
    o_ref[...] = jnp.max(x, axis=1)[None]


def _row_spec(c):
    return pl.BlockSpec((RCHUNK, c), lambda s, i: (s * NCHUNK + i, 0))


def _seg_spec(shape):
    return pl.BlockSpec((1,) + shape[1:], lambda s, i: (s,) + (0,) * (len(shape) - 1))


def _stat_spec(c):
    return pl.BlockSpec((1, 1, c), lambda s, i: (s, 0, 0))


def _mlp(gath, nx, lxyz, w, b, g, be):
    grid = (NSEG, NCHUNK)
    y0, s0, q0 = pl.pallas_call(
        _l0_body,
        grid=grid,
        in_specs=[_row_spec(NCH), _row_spec(NCH),
                  _seg_spec(w[0].shape), _seg_spec(b[0].shape)],
        out_specs=[_row_spec(64), _stat_spec(64), _stat_spec(64)],
        out_shape=[jax.ShapeDtypeStruct((NROW, 64), jnp.float32),
                   jax.ShapeDtypeStruct((NSEG, 1, 64), jnp.float32),
                   jax.ShapeDtypeStruct((NSEG, 1, 64), jnp.float32)],
    )(gath, nx, w[0], b[0])

    def layer(yk, sk, qk, l, co):
        return pl.pallas_call(
            _lk_body,
            grid=grid,
            in_specs=[_row_spec(yk.shape[-1]),
                      _stat_spec(yk.shape[-1]), _stat_spec(yk.shape[-1]),
                      _seg_spec(g[l - 1].shape), _seg_spec(be[l - 1].shape),
                      _seg_spec(w[l].shape), _seg_spec(b[l].shape)],
            out_specs=[_row_spec(co), _stat_spec(co), _stat_spec(co)],
            out_shape=[jax.ShapeDtypeStruct((NROW, co), jnp.float32),
                       jax.ShapeDtypeStruct((NSEG, 1, co), jnp.float32),
                       jax.ShapeDtypeStruct((NSEG, 1, co), jnp.float32)],
        )(yk, sk, qk, g[l - 1], be[l - 1], w[l], b[l])

    y1, s1, q1 = layer(y0, s0, q0, 1, 64)
    y2, s2, q2 = layer(y1, s1, q1, 2, 128)

    out = pl.pallas_call(
        _fin_body,
        grid=grid,
        in_specs=[_row_spec(128), _stat_spec(128), _stat_spec(128),
                  _seg_spec(g[2].shape), _seg_spec(be[2].shape)],
        out_specs=pl.BlockSpec((1, RCHUNK // NSAMPLE, 128),
                               lambda s, i: (s, i, 0)),
        out_shape=jax.ShapeDtypeStruct((NSEG, NPOINT, 128), jnp.float32),
    )(y2, s2, q2, g[2], be[2])
    return out


def kernel(xy, xyz, points, point_split, conv_w0, conv_b0, conv_w1, conv_b1,
           conv_w2, conv_b2, bn_g0, bn_b0, bn_g1, bn_b1, bn_g2, bn_b2):
    xy = xy + point_split[0].astype(xy.dtype)
    fps = _fps(xyz)

    xyz2 = xyz[0].reshape(3, NSEG, NPT).transpose(1, 0, 2)
    table = jnp.concatenate([xyz[0].T, points[0].T], axis=1)
    g16 = jnp.concatenate([xyz[0].T, xy[0].T,
                           jnp.zeros((NSEG * NPT, 11), jnp.float32)], axis=1)
    gath, lc = _sc_stage(xyz2.reshape(-1), fps.reshape(-1), g16, table)
    lc = lc.reshape(NSEG, NPOINT, 16)
    lxyz = lc[:, :, 0:3].transpose(0, 2, 1)
    lxy = lc[:, :, 3:5].transpose(0, 2, 1)

    nx3 = jnp.broadcast_to(lxyz.transpose(0, 2, 1)[:, :, None, :],
                           (NSEG, NPOINT, NSAMPLE, 3)).reshape(NROW, 3)
    nx = jnp.concatenate(
        [nx3, jnp.zeros((NROW, NCH - 3), jnp.float32)], axis=1)

    w = (conv_w0, conv_w1, conv_w2)
    b = tuple(x.reshape(NSEG, 1, -1) for x in (conv_b0, conv_b1, conv_b2))
    g = tuple(x.reshape(NSEG, 1, -1) for x in (bn_g0, bn_g1, bn_g2))
    be = tuple(x.reshape(NSEG, 1, -1) for x in (bn_b0, bn_b1, bn_b2))
    out = _mlp(gath, nx, lxyz, w, b, g, be)

    new_xy = jnp.concatenate([lxy[0], lxy[1]], axis=-1)[None]
    new_xyz = jnp.concatenate([lxyz[0], lxyz[1]], axis=-1)[None]
    new_pts = jnp.concatenate([out[0].T, out[1].T], axis=-1)[None]
    split = jnp.array([0, NPOINT, 2 * NPOINT], dtype=jnp.int32)
    return new_xy, new_xyz, new_pts, split
